# Initial kernel scaffold; baseline (speedup 1.0000x reference)
import jax
import jax.numpy as jnp
from jax.experimental import pallas as pl


def kernel(e, v, edges, W, b):
    raise NotImplementedError("write your pallas kernel here")


# speedup vs baseline: 4.1919x; 4.1919x over previous
import functools

import jax
import jax.numpy as jnp
from jax import lax
from jax.experimental import pallas as pl
from jax.experimental.pallas import tpu as pltpu
from jax.experimental.pallas import tpu_sc as plsc

N_NODES = 10000
N_EDGES = 320000
D_FEAT = 128
D_EDGE = 16
D_OUT = 16

_PACK = 128 // D_EDGE
_EB_ROWS = N_EDGES // _PACK
_BLK = 4000

_NC, _NS = 2, 16
_NW = _NC * _NS
_PER_W = N_EDGES // _NW
_CH = 1024
_SUB = 128
_NFULL = _PER_W // _CH
_TAIL = _PER_W - _NFULL * _CH


def _proj_body(v_ref, w1_ref, w2_ref, p_ref, q_ref):
    v = v_ref[...]
    p_ref[...] = jnp.dot(v, w1_ref[...], preferred_element_type=jnp.float32)
    q_ref[...] = jnp.dot(v, w2_ref[...], preferred_element_type=jnp.float32)


def _base_body(eb_ref, wbd_ref, bt_ref, o_ref):
    o_ref[...] = (
        jnp.dot(eb_ref[...], wbd_ref[...], preferred_element_type=jnp.float32)
        + bt_ref[...]
    )


def _sc_body(src_hbm, dst_hbm, p_hbm, q_hbm, base_hbm, out_hbm,
             idx_s, idx_d, rows_s, rows_d, acc, sem):
    wid = lax.axis_index("s") * _NC + lax.axis_index("c")
    wb = wid * _PER_W

    def do_chunk(off, n):
        pltpu.sync_copy(src_hbm.at[pl.ds(off, n)], idx_s.at[pl.ds(0, n)])
        pltpu.sync_copy(dst_hbm.at[pl.ds(off, n)], idx_d.at[pl.ds(0, n)])
        descs = []
        s0 = 0
        while s0 < n:
            sz = min(_SUB, n - s0)
            descs.append(pltpu.async_copy(
                p_hbm.at[idx_s.at[pl.ds(s0, sz)]],
                rows_s.at[pl.ds(s0, sz)], sem))
            descs.append(pltpu.async_copy(
                q_hbm.at[idx_d.at[pl.ds(s0, sz)]],
                rows_d.at[pl.ds(s0, sz)], sem))
            s0 += sz
        pltpu.sync_copy(base_hbm.at[pl.ds(off, n)], acc.at[pl.ds(0, n)])
        for d in descs:
            d.wait()

        def add_row(i, c):
            acc[i, :] = acc[i, :] + rows_s[i, :] + rows_d[i, :]
            return c

        lax.fori_loop(0, n, add_row, 0)
        pltpu.sync_copy(acc.at[pl.ds(0, n)], out_hbm.at[pl.ds(off, n)])

    def chunk_loop(c, carry):
        do_chunk(wb + c * _CH, _CH)
        return carry

    lax.fori_loop(0, _NFULL, chunk_loop, 0)
    do_chunk(wb + _NFULL * _CH, _TAIL)


def kernel(e, v, edges, W, b):
    W1 = W[:D_FEAT]
    W2 = W[D_FEAT:2 * D_FEAT]
    We = W[2 * D_FEAT:]
    Wbd = jnp.kron(jnp.eye(_PACK, dtype=W.dtype), We)
    btile = jnp.tile(b, _PACK).reshape(1, 128)
    src = edges[:, 0]
    dst = edges[:, 1]

    P, Q = pl.pallas_call(
        _proj_body,
        out_shape=[jax.ShapeDtypeStruct((N_NODES, D_OUT), jnp.float32)] * 2,
    )(v, W1, W2)

    eb = e.reshape(_EB_ROWS, 128)
    base = pl.pallas_call(
        _base_body,
        grid=(_EB_ROWS // _BLK,),
        in_specs=[
            pl.BlockSpec((_BLK, 128), lambda i: (i, 0)),
            pl.BlockSpec((128, 128), lambda i: (0, 0)),
            pl.BlockSpec((1, 128), lambda i: (0, 0)),
        ],
        out_specs=pl.BlockSpec((_BLK, 128), lambda i: (i, 0)),
        out_shape=jax.ShapeDtypeStruct((_EB_ROWS, 128), jnp.float32),
    )(eb, Wbd, btile)
    base = base.reshape(N_EDGES, D_OUT)

    mesh = plsc.VectorSubcoreMesh(
        core_axis_name="c", subcore_axis_name="s",
        num_cores=_NC, num_subcores=_NS)
    sc = pl.kernel(
        _sc_body,
        out_type=jax.ShapeDtypeStruct((N_EDGES, D_OUT), jnp.float32),
        mesh=mesh,
        scratch_types=[
            pltpu.VMEM((_CH,), jnp.int32),
            pltpu.VMEM((_CH,), jnp.int32),
            pltpu.VMEM((_CH, D_OUT), jnp.float32),
            pltpu.VMEM((_CH, D_OUT), jnp.float32),
            pltpu.VMEM((_CH, D_OUT), jnp.float32),
            pltpu.SemaphoreType.DMA,
        ],
        compiler_params=pltpu.CompilerParams(use_tc_tiling_on_sc=False),
    )
    return sc(src, dst, P, Q, base)


# speedup vs baseline: 6.6802x; 1.5936x over previous
import functools

import jax
import jax.numpy as jnp
from jax import lax
from jax.experimental import pallas as pl
from jax.experimental.pallas import tpu as pltpu
from jax.experimental.pallas import tpu_sc as plsc

N_NODES = 10000
N_EDGES = 320000
D_FEAT = 128
D_EDGE = 16
D_OUT = 16

_NT = N_EDGES // 128
_BLK = 16000
_BT = _BLK // 128

_NC, _NS = 2, 16
_NW = _NC * _NS
_CT = 10
_CH = _CT * 128
_NCHUNK = N_EDGES // _CH
_SUB = 128


def _proj_body(v_ref, w1_ref, w2_ref, p_ref, q_ref):
    v = v_ref[...]
    p_ref[...] = jnp.dot(v, w1_ref[...], preferred_element_type=jnp.float32)
    q_ref[...] = jnp.dot(v, w2_ref[...], preferred_element_type=jnp.float32)


def _base_body(eT_ref, weT_ref, bT_ref, o_ref):
    m = jnp.dot(weT_ref[...], eT_ref[...],
                preferred_element_type=jnp.float32) + bT_ref[...]
    x = m.reshape(2, 8, _BT, 128)
    o_ref[...] = x.transpose(0, 2, 1, 3)


def _sc_body(src_hbm, dst_hbm, p_hbm, q_hbm, base_hbm, out_hbm,
             idx_s, idx_d, rows_s, rows_d, acc, sem):
    wid = lax.axis_index("s") * _NC + lax.axis_index("c")
    ii = lax.iota(jnp.int32, 16)
    f_hi = lax.shift_right_logical(ii, 3)
    f_lo = lax.bitwise_and(ii, 7)

    def do_chunk(c, carry):
        chunk = wid + c * _NW
        off = chunk * _CH
        t0 = chunk * _CT
        pltpu.sync_copy(src_hbm.at[pl.ds(off, _CH)], idx_s)
        pltpu.sync_copy(dst_hbm.at[pl.ds(off, _CH)], idx_d)
        descs = []
        for k in range(_CH // _SUB):
            s0 = k * _SUB
            descs.append(pltpu.async_copy(
                p_hbm.at[idx_s.at[pl.ds(s0, _SUB)]],
                rows_s.at[pl.ds(s0, _SUB)], sem))
            descs.append(pltpu.async_copy(
                q_hbm.at[idx_d.at[pl.ds(s0, _SUB)]],
                rows_d.at[pl.ds(s0, _SUB)], sem))
        pltpu.sync_copy(base_hbm.at[0, pl.ds(t0, _CT)], acc.at[0])
        pltpu.sync_copy(base_hbm.at[1, pl.ds(t0, _CT)], acc.at[1])
        for d in descs:
            d.wait()

        def add_row(i, cc):
            q = lax.shift_right_logical(i, 7)
            r = lax.bitwise_and(i, 127)
            x = rows_s[i, :] + rows_d[i, :]
            plsc.addupdate_scatter(
                acc,
                [f_hi, jnp.broadcast_to(q, (16,)), f_lo,
                 jnp.broadcast_to(r, (16,))],
                x)
            return cc

        lax.fori_loop(0, _CH, add_row, 0)
        pltpu.sync_copy(acc.at[0], out_hbm.at[0, pl.ds(t0, _CT)])
        pltpu.sync_copy(acc.at[1], out_hbm.at[1, pl.ds(t0, _CT)])
        return carry

    nc = (_NCHUNK - wid + _NW - 1) // _NW
    lax.fori_loop(0, nc, do_chunk, 0)


def kernel(e, v, edges, W, b):
    W1 = W[:D_FEAT]
    W2 = W[D_FEAT:2 * D_FEAT]
    WeT = W[2 * D_FEAT:].T
    bT = b.reshape(D_OUT, 1)
    eT = e.T
    src = edges[:, 0]
    dst = edges[:, 1]

    P, Q = pl.pallas_call(
        _proj_body,
        out_shape=[jax.ShapeDtypeStruct((N_NODES, D_OUT), jnp.float32)] * 2,
    )(v, W1, W2)

    base4 = pl.pallas_call(
        _base_body,
        grid=(N_EDGES // _BLK,),
        in_specs=[
            pl.BlockSpec((D_OUT, _BLK), lambda i: (0, i)),
            pl.BlockSpec((D_OUT, D_OUT), lambda i: (0, 0)),
            pl.BlockSpec((D_OUT, 1), lambda i: (0, 0)),
        ],
        out_specs=pl.BlockSpec((2, _BT, 8, 128), lambda i: (0, i, 0, 0)),
        out_shape=jax.ShapeDtypeStruct((2, _NT, 8, 128), jnp.float32),
    )(eT, WeT, bT)

    mesh = plsc.VectorSubcoreMesh(
        core_axis_name="c", subcore_axis_name="s",
        num_cores=_NC, num_subcores=_NS)
    sc = pl.kernel(
        _sc_body,
        out_type=jax.ShapeDtypeStruct((2, _NT, 8, 128), jnp.float32),
        mesh=mesh,
        scratch_types=[
            pltpu.VMEM((_CH,), jnp.int32),
            pltpu.VMEM((_CH,), jnp.int32),
            pltpu.VMEM((_CH, D_OUT), jnp.float32),
            pltpu.VMEM((_CH, D_OUT), jnp.float32),
            pltpu.VMEM((2, _CT, 8, 128), jnp.float32),
            pltpu.SemaphoreType.DMA,
        ],
        compiler_params=pltpu.CompilerParams(
            use_tc_tiling_on_sc=False, needs_layout_passes=False),
    )
    out4 = sc(src, dst, P, Q, base4)
    return out4.transpose(1, 3, 0, 2).reshape(N_EDGES, D_OUT)


# speedup vs baseline: 8.4849x; 1.2702x over previous
import functools

import jax
import jax.numpy as jnp
from jax import lax
from jax.experimental import pallas as pl
from jax.experimental.pallas import tpu as pltpu
from jax.experimental.pallas import tpu_sc as plsc

N_NODES = 10000
N_EDGES = 320000
D_FEAT = 128
D_EDGE = 16
D_OUT = 16

_NT = N_EDGES // 128
_BLK = 16000
_BT = _BLK // 128

_NC, _NS = 2, 16
_NW = _NC * _NS
_CT = 10
_CH = _CT * 128
_NCHUNK = N_EDGES // _CH
_SUB = 128


def _proj_body(v_ref, w1_ref, w2_ref, p_ref, q_ref):
    v = v_ref[...]
    p_ref[...] = jnp.dot(v, w1_ref[...], preferred_element_type=jnp.float32)
    q_ref[...] = jnp.dot(v, w2_ref[...], preferred_element_type=jnp.float32)


def _base_body(eT_ref, weT_ref, bT_ref, o_ref):
    m = jnp.dot(weT_ref[...], eT_ref[...],
                preferred_element_type=jnp.float32) + bT_ref[...]
    x = m.reshape(2, 8, _BT, 128)
    o_ref[...] = x.transpose(0, 2, 1, 3)


def _sc_body(ed_hbm, p_hbm, q_hbm, base_hbm, out_hbm,
             ed_v, rows_s, rows_d, acc, sem):
    wid = lax.axis_index("s") * _NC + lax.axis_index("c")
    ii = lax.iota(jnp.int32, 16)
    c1 = (lax.shift_right_logical(ii, 3) * (_CT * 1024)
          + lax.bitwise_and(ii, 7) * 128)
    half = _NT * 1024

    def do_chunk(c, carry):
        chunk = wid + c * _NW
        t0 = chunk * _CT
        pltpu.sync_copy(ed_hbm.at[pl.ds(t0, _CT)], ed_v)
        descs = []
        for k in range(_CT):
            s0 = k * _SUB
            descs.append(pltpu.async_copy(
                p_hbm.at[ed_v.at[k, 0]],
                rows_s.at[pl.ds(s0, _SUB)], sem))
            descs.append(pltpu.async_copy(
                q_hbm.at[ed_v.at[k, 1]],
                rows_d.at[pl.ds(s0, _SUB)], sem))
        pltpu.sync_copy(base_hbm.at[pl.ds(t0 * 1024, _CT * 1024)],
                        acc.at[pl.ds(0, _CT * 1024)])
        pltpu.sync_copy(base_hbm.at[pl.ds(half + t0 * 1024, _CT * 1024)],
                        acc.at[pl.ds(_CT * 1024, _CT * 1024)])
        for d in descs:
            d.wait()

        @plsc.parallel_loop(0, _CH, 1, unroll=8)
        def add_row(i):
            x = rows_s[i, :] + rows_d[i, :]
            slot = (lax.shift_left(lax.shift_right_logical(i, 7), 10)
                    + lax.bitwise_and(i, 127))
            plsc.addupdate_scatter(acc, [c1 + slot], x)

        pltpu.sync_copy(acc.at[pl.ds(0, _CT * 1024)],
                        out_hbm.at[pl.ds(t0 * 1024, _CT * 1024)])
        pltpu.sync_copy(acc.at[pl.ds(_CT * 1024, _CT * 1024)],
                        out_hbm.at[pl.ds(half + t0 * 1024, _CT * 1024)])
        return carry

    nc = (_NCHUNK - wid + _NW - 1) // _NW
    lax.fori_loop(0, nc, do_chunk, 0)


def kernel(e, v, edges, W, b):
    W1 = W[:D_FEAT]
    W2 = W[D_FEAT:2 * D_FEAT]
    WeT = W[2 * D_FEAT:].T
    bT = b.reshape(D_OUT, 1)
    eT = e.T
    ed3 = edges.T.reshape(2, _NT, 128).transpose(1, 0, 2)

    P, Q = pl.pallas_call(
        _proj_body,
        out_shape=[jax.ShapeDtypeStruct((N_NODES, D_OUT), jnp.float32)] * 2,
    )(v, W1, W2)

    base4 = pl.pallas_call(
        _base_body,
        grid=(N_EDGES // _BLK,),
        in_specs=[
            pl.BlockSpec((D_OUT, _BLK), lambda i: (0, i)),
            pl.BlockSpec((D_OUT, D_OUT), lambda i: (0, 0)),
            pl.BlockSpec((D_OUT, 1), lambda i: (0, 0)),
        ],
        out_specs=pl.BlockSpec((2, _BT, 8, 128), lambda i: (0, i, 0, 0)),
        out_shape=jax.ShapeDtypeStruct((2, _NT, 8, 128), jnp.float32),
    )(eT, WeT, bT)

    mesh = plsc.VectorSubcoreMesh(
        core_axis_name="c", subcore_axis_name="s",
        num_cores=_NC, num_subcores=_NS)
    sc = pl.kernel(
        _sc_body,
        out_type=jax.ShapeDtypeStruct((2 * _NT * 8 * 128,), jnp.float32),
        mesh=mesh,
        scratch_types=[
            pltpu.VMEM((_CT, 2, 128), jnp.int32),
            pltpu.VMEM((_CH, D_OUT), jnp.float32),
            pltpu.VMEM((_CH, D_OUT), jnp.float32),
            pltpu.VMEM((2 * _CT * 8 * 128,), jnp.float32),
            pltpu.SemaphoreType.DMA,
        ],
        compiler_params=pltpu.CompilerParams(
            use_tc_tiling_on_sc=False, needs_layout_passes=False),
    )
    out_flat = sc(ed3, P, Q, base4.reshape(-1))
    return (out_flat.reshape(2, _NT, 8, 128)
            .transpose(1, 3, 0, 2).reshape(N_EDGES, D_OUT))


# speedup vs baseline: 8.6045x; 1.0141x over previous
import functools

import jax
import jax.numpy as jnp
from jax import lax
from jax.experimental import pallas as pl
from jax.experimental.pallas import tpu as pltpu
from jax.experimental.pallas import tpu_sc as plsc

N_NODES = 10000
N_EDGES = 320000
D_FEAT = 128
D_EDGE = 16
D_OUT = 16

_NT = N_EDGES // 128
_BLK = 16000
_BT = _BLK // 128

_NC, _NS = 2, 16
_NW = _NC * _NS
_CT = 5
_CH = _CT * 128
_NCHUNK = N_EDGES // _CH
_NSLOT = -(-_NCHUNK // _NW)
_SUB = 128


def _proj_body(v_ref, w1_ref, w2_ref, p_ref, q_ref):
    v = v_ref[...]
    p_ref[...] = jnp.dot(v, w1_ref[...], preferred_element_type=jnp.float32)
    q_ref[...] = jnp.dot(v, w2_ref[...], preferred_element_type=jnp.float32)


def _base_body(eT_ref, weT_ref, bT_ref, o_ref):
    m = jnp.dot(weT_ref[...], eT_ref[...],
                preferred_element_type=jnp.float32) + bT_ref[...]
    x = m.reshape(2, 8, _BT, 128)
    o_ref[...] = x.transpose(0, 2, 1, 3)


def _sc_body(ed_hbm, p_hbm, q_hbm, base_hbm, out_hbm,
             ed_v, rows_s, rows_d, acc,
             sem_g0, sem_g1, sem_o0, sem_o1):
    wid = lax.axis_index("s") * _NC + lax.axis_index("c")
    ii = lax.iota(jnp.int32, 16)
    c1 = (lax.shift_right_logical(ii, 3) * (_CT * 1024)
          + lax.bitwise_and(ii, 7) * 128)
    half = _NT * 1024
    sem_g = (sem_g0, sem_g1)
    sem_o = (sem_o0, sem_o1)
    cw = _CT * 1024

    def issue_gathers(chunk, buf):
        pltpu.sync_copy(ed_hbm.at[pl.ds(chunk * _CT, _CT)], ed_v.at[buf])
        for k in range(_CT):
            s0 = k * _SUB
            pltpu.async_copy(p_hbm.at[ed_v.at[buf, k, 0]],
                             rows_s.at[buf, pl.ds(s0, _SUB)], sem_g[buf])
            pltpu.async_copy(q_hbm.at[ed_v.at[buf, k, 1]],
                             rows_d.at[buf, pl.ds(s0, _SUB)], sem_g[buf])

    def drain_gathers(buf):
        for k in range(_CT):
            s0 = k * _SUB
            pltpu.make_async_copy(p_hbm.at[pl.ds(0, _SUB)],
                                  rows_s.at[buf, pl.ds(s0, _SUB)],
                                  sem_g[buf]).wait()
            pltpu.make_async_copy(q_hbm.at[pl.ds(0, _SUB)],
                                  rows_d.at[buf, pl.ds(s0, _SUB)],
                                  sem_g[buf]).wait()

    def drain_out(buf):
        pltpu.make_async_copy(base_hbm.at[pl.ds(0, cw)],
                              acc.at[buf, pl.ds(0, cw)], sem_o[buf]).wait()
        pltpu.make_async_copy(base_hbm.at[pl.ds(0, cw)],
                              acc.at[buf, pl.ds(cw, cw)], sem_o[buf]).wait()

    def do_half(k, buf):
        chunk = wid + k * _NW

        @pl.when(chunk < _NCHUNK)
        def _():
            drain_gathers(buf)

            @pl.when(k >= 2)
            def _():
                drain_out(buf)

            nxt = chunk + _NW

            @pl.when(nxt < _NCHUNK)
            def _():
                issue_gathers(nxt, buf ^ 1)

            t0 = chunk * _CT
            pltpu.sync_copy(base_hbm.at[pl.ds(t0 * 1024, cw)],
                            acc.at[buf, pl.ds(0, cw)])
            pltpu.sync_copy(base_hbm.at[pl.ds(half + t0 * 1024, cw)],
                            acc.at[buf, pl.ds(cw, cw)])

            @plsc.parallel_loop(0, _CH, 1, unroll=8)
            def add_row(i):
                x = rows_s[buf, i, :] + rows_d[buf, i, :]
                slot = (lax.shift_left(lax.shift_right_logical(i, 7), 10)
                        + lax.bitwise_and(i, 127))
                plsc.addupdate_scatter(acc.at[buf], [c1 + slot], x)

            pltpu.async_copy(acc.at[buf, pl.ds(0, cw)],
                             out_hbm.at[pl.ds(t0 * 1024, cw)], sem_o[buf])
            pltpu.async_copy(acc.at[buf, pl.ds(cw, cw)],
                             out_hbm.at[pl.ds(half + t0 * 1024, cw)],
                             sem_o[buf])

    issue_gathers(wid, 0)

    def pair(j, carry):
        do_half(2 * j, 0)
        do_half(2 * j + 1, 1)
        return carry

    lax.fori_loop(0, _NSLOT // 2, pair, 0)

    drain_out(0)
    drain_out(1)


def kernel(e, v, edges, W, b):
    W1 = W[:D_FEAT]
    W2 = W[D_FEAT:2 * D_FEAT]
    WeT = W[2 * D_FEAT:].T
    bT = b.reshape(D_OUT, 1)
    eT = e.T
    ed3 = edges.T.reshape(2, _NT, 128).transpose(1, 0, 2)

    P, Q = pl.pallas_call(
        _proj_body,
        out_shape=[jax.ShapeDtypeStruct((N_NODES, D_OUT), jnp.float32)] * 2,
    )(v, W1, W2)

    base4 = pl.pallas_call(
        _base_body,
        grid=(N_EDGES // _BLK,),
        in_specs=[
            pl.BlockSpec((D_OUT, _BLK), lambda i: (0, i)),
            pl.BlockSpec((D_OUT, D_OUT), lambda i: (0, 0)),
            pl.BlockSpec((D_OUT, 1), lambda i: (0, 0)),
        ],
        out_specs=pl.BlockSpec((2, _BT, 8, 128), lambda i: (0, i, 0, 0)),
        out_shape=jax.ShapeDtypeStruct((2, _NT, 8, 128), jnp.float32),
    )(eT, WeT, bT)

    mesh = plsc.VectorSubcoreMesh(
        core_axis_name="c", subcore_axis_name="s",
        num_cores=_NC, num_subcores=_NS)
    sc = pl.kernel(
        _sc_body,
        out_type=jax.ShapeDtypeStruct((2 * _NT * 8 * 128,), jnp.float32),
        mesh=mesh,
        scratch_types=[
            pltpu.VMEM((2, _CT, 2, 128), jnp.int32),
            pltpu.VMEM((2, _CH, D_OUT), jnp.float32),
            pltpu.VMEM((2, _CH, D_OUT), jnp.float32),
            pltpu.VMEM((2, 2 * _CT * 8 * 128), jnp.float32),
            pltpu.SemaphoreType.DMA,
            pltpu.SemaphoreType.DMA,
            pltpu.SemaphoreType.DMA,
            pltpu.SemaphoreType.DMA,
        ],
        compiler_params=pltpu.CompilerParams(
            use_tc_tiling_on_sc=False, needs_layout_passes=False),
    )
    out_flat = sc(ed3, P, Q, base4.reshape(-1))
    return (out_flat.reshape(2, _NT, 8, 128)
            .transpose(1, 3, 0, 2).reshape(N_EDGES, D_OUT))


# speedup vs baseline: 8.7541x; 1.0174x over previous
import functools

import jax
import jax.numpy as jnp
from jax import lax
from jax.experimental import pallas as pl
from jax.experimental.pallas import tpu as pltpu
from jax.experimental.pallas import tpu_sc as plsc

N_NODES = 10000
N_EDGES = 320000
D_FEAT = 128
D_EDGE = 16
D_OUT = 16

_NT = N_EDGES // 128
_BLK = 16000
_BT = _BLK // 128

_NC, _NS = 2, 16
_NW = _NC * _NS
_CT = 10
_CH = _CT * 128
_NCHUNK = N_EDGES // _CH
_NSLOT = -(-_NCHUNK // _NW)


def _proj_body(v_ref, w1_ref, w2_ref, p_ref, q_ref):
    v = v_ref[...]
    p_ref[...] = jnp.dot(v, w1_ref[...], preferred_element_type=jnp.float32)
    q_ref[...] = jnp.dot(v, w2_ref[...], preferred_element_type=jnp.float32)


def _base_body(eT_ref, weT_ref, bT_ref, o_ref):
    m = jnp.dot(weT_ref[...], eT_ref[...],
                preferred_element_type=jnp.float32) + bT_ref[...]
    x = m.reshape(2, 8, _BT, 128)
    o_ref[...] = x.transpose(0, 2, 1, 3)


def _sc_body(sd_hbm, p_hbm, q_hbm, base_hbm, out_hbm,
             idx_s, idx_d, rows_s, rows_d, acc,
             sem_g0, sem_g1, sem_o):
    wid = lax.axis_index("s") * _NC + lax.axis_index("c")
    ii = lax.iota(jnp.int32, 16)
    c1 = (lax.shift_right_logical(ii, 3) * (_CT * 1024)
          + lax.bitwise_and(ii, 7) * 128)
    half = _NT * 1024
    sem_g = (sem_g0, sem_g1)
    cw = _CT * 1024

    def issue_gathers(chunk, buf):
        off = chunk * _CH
        pltpu.sync_copy(sd_hbm.at[0, pl.ds(off, _CH)], idx_s.at[buf])
        pltpu.sync_copy(sd_hbm.at[1, pl.ds(off, _CH)], idx_d.at[buf])
        pltpu.async_copy(p_hbm.at[idx_s.at[buf]], rows_s.at[buf], sem_g[buf])
        pltpu.async_copy(q_hbm.at[idx_d.at[buf]], rows_d.at[buf], sem_g[buf])

    def drain_gathers(buf):
        pltpu.make_async_copy(p_hbm.at[pl.ds(0, _CH)],
                              rows_s.at[buf], sem_g[buf]).wait()
        pltpu.make_async_copy(q_hbm.at[pl.ds(0, _CH)],
                              rows_d.at[buf], sem_g[buf]).wait()

    def drain_out():
        pltpu.make_async_copy(base_hbm.at[pl.ds(0, cw)],
                              acc.at[pl.ds(0, cw)], sem_o).wait()
        pltpu.make_async_copy(base_hbm.at[pl.ds(0, cw)],
                              acc.at[pl.ds(cw, cw)], sem_o).wait()

    def do_half(k, buf):
        chunk = wid + k * _NW

        @pl.when(chunk < _NCHUNK)
        def _():
            nxt = chunk + _NW

            @pl.when(nxt < _NCHUNK)
            def _():
                issue_gathers(nxt, buf ^ 1)

            @pl.when(k >= 1)
            def _():
                drain_out()

            t0 = chunk * _CT
            pltpu.sync_copy(base_hbm.at[pl.ds(t0 * 1024, cw)],
                            acc.at[pl.ds(0, cw)])
            pltpu.sync_copy(base_hbm.at[pl.ds(half + t0 * 1024, cw)],
                            acc.at[pl.ds(cw, cw)])
            drain_gathers(buf)

            @plsc.parallel_loop(0, _CH, 1, unroll=8)
            def add_row(i):
                x = rows_s[buf, i, :] + rows_d[buf, i, :]
                slot = (lax.shift_left(lax.shift_right_logical(i, 7), 10)
                        + lax.bitwise_and(i, 127))
                plsc.addupdate_scatter(acc, [c1 + slot], x)

            pltpu.async_copy(acc.at[pl.ds(0, cw)],
                             out_hbm.at[pl.ds(t0 * 1024, cw)], sem_o)
            pltpu.async_copy(acc.at[pl.ds(cw, cw)],
                             out_hbm.at[pl.ds(half + t0 * 1024, cw)], sem_o)

    issue_gathers(wid, 0)

    def pair(j, carry):
        do_half(2 * j, 0)
        do_half(2 * j + 1, 1)
        return carry

    lax.fori_loop(0, _NSLOT // 2, pair, 0)
    drain_out()


def kernel(e, v, edges, W, b):
    W1 = W[:D_FEAT]
    W2 = W[D_FEAT:2 * D_FEAT]
    WeT = W[2 * D_FEAT:].T
    bT = b.reshape(D_OUT, 1)
    eT = e.T
    sd = edges.T

    P, Q = pl.pallas_call(
        _proj_body,
        out_shape=[jax.ShapeDtypeStruct((N_NODES, D_OUT), jnp.float32)] * 2,
    )(v, W1, W2)

    base4 = pl.pallas_call(
        _base_body,
        grid=(N_EDGES // _BLK,),
        in_specs=[
            pl.BlockSpec((D_OUT, _BLK), lambda i: (0, i)),
            pl.BlockSpec((D_OUT, D_OUT), lambda i: (0, 0)),
            pl.BlockSpec((D_OUT, 1), lambda i: (0, 0)),
        ],
        out_specs=pl.BlockSpec((2, _BT, 8, 128), lambda i: (0, i, 0, 0)),
        out_shape=jax.ShapeDtypeStruct((2, _NT, 8, 128), jnp.float32),
    )(eT, WeT, bT)

    mesh = plsc.VectorSubcoreMesh(
        core_axis_name="c", subcore_axis_name="s",
        num_cores=_NC, num_subcores=_NS)
    sc = pl.kernel(
        _sc_body,
        out_type=jax.ShapeDtypeStruct((2 * _NT * 8 * 128,), jnp.float32),
        mesh=mesh,
        scratch_types=[
            pltpu.VMEM((2, _CH), jnp.int32),
            pltpu.VMEM((2, _CH), jnp.int32),
            pltpu.VMEM((2, _CH, D_OUT), jnp.float32),
            pltpu.VMEM((2, _CH, D_OUT), jnp.float32),
            pltpu.VMEM((2 * _CT * 8 * 128,), jnp.float32),
            pltpu.SemaphoreType.DMA,
            pltpu.SemaphoreType.DMA,
            pltpu.SemaphoreType.DMA,
        ],
        compiler_params=pltpu.CompilerParams(
            use_tc_tiling_on_sc=False, needs_layout_passes=False),
    )
    out_flat = sc(sd, P, Q, base4.reshape(-1))
    return (out_flat.reshape(2, _NT, 8, 128)
            .transpose(1, 3, 0, 2).reshape(N_EDGES, D_OUT))
